# Initial kernel scaffold; baseline (speedup 1.0000x reference)
import jax
import jax.numpy as jnp
from jax.experimental import pallas as pl


def kernel(x, edge_index, Wi, bi, W0, as0, ad0, bb0, g0, be0, W1, as1, ad1, bb1, g1, be1, W2, as2, ad2, bb2, g2, be2, Wo, bo):
    raise NotImplementedError("write your pallas kernel here")


# speedup vs baseline: 1.0000x; 1.0000x over previous
import functools

import jax
import jax.numpy as jnp
from jax.experimental import pallas as pl
from jax.experimental.pallas import tpu as pltpu

N = 10000
E = 160000
D = 256
H = 8
C = 32


def _proj_body(x_ref, w_ref, b_ref, o_ref):
    o_ref[...] = jax.nn.relu(
        jnp.dot(x_ref[...], w_ref[...], preferred_element_type=jnp.float32)
        + b_ref[...]
    )


def _proj(x, W, b):
    blk = 1000
    return pl.pallas_call(
        _proj_body,
        grid=(N // blk,),
        in_specs=[
            pl.BlockSpec((blk, D), lambda i: (i, 0)),
            pl.BlockSpec((D, D), lambda i: (0, 0)),
            pl.BlockSpec((1, D), lambda i: (0, 0)),
        ],
        out_specs=pl.BlockSpec((blk, D), lambda i: (i, 0)),
        out_shape=jax.ShapeDtypeStruct((N, D), jnp.float32),
    )(x, W, b.reshape(1, D))


def _layer_norm(x, g, b):
    mu = jnp.mean(x, axis=-1, keepdims=True)
    var = jnp.var(x, axis=-1, keepdims=True)
    return (x - mu) / jnp.sqrt(var + 1e-5) * g + b


def _gat(h, src, dst, W, a_s, a_d, b):
    xp = (h @ W).reshape(-1, H, C)
    als = jnp.sum(xp * a_s, axis=-1)
    ald = jnp.sum(xp * a_d, axis=-1)
    e = jax.nn.leaky_relu(als[src] + ald[dst], 0.2)
    emax = jax.ops.segment_max(e, dst, num_segments=N)
    emax = jnp.where(jnp.isfinite(emax), emax, 0.0)
    ex = jnp.exp(e - emax[dst])
    den = jax.ops.segment_sum(ex, dst, num_segments=N)
    alpha = ex / (den[dst] + 1e-16)
    out = jax.ops.segment_sum(xp[src] * alpha[:, :, None], dst, num_segments=N)
    return out.reshape(-1, H * C) + b


def kernel(x, edge_index, Wi, bi, W0, as0, ad0, bb0, g0, be0, W1, as1, ad1, bb1, g1, be1, W2, as2, ad2, bb2, g2, be2, Wo, bo):
    loops = jnp.arange(N, dtype=edge_index.dtype)
    src = jnp.concatenate([edge_index[0], loops])
    dst = jnp.concatenate([edge_index[1], loops])
    h = _proj(x, Wi, bi)
    params = [(W0, as0, ad0, bb0, g0, be0), (W1, as1, ad1, bb1, g1, be1), (W2, as2, ad2, bb2, g2, be2)]
    for (W, a_s, a_d, bb, g, be) in params:
        hn = _gat(h, src, dst, W, a_s, a_d, bb)
        hn = _layer_norm(hn, g, be)
        hn = jax.nn.relu(hn)
        h = h + hn
    return h @ Wo + bo


# speedup vs baseline: 14.1988x; 14.1986x over previous
import functools

import jax
import jax.numpy as jnp
from jax import lax
from jax.experimental import pallas as pl
from jax.experimental.pallas import tpu as pltpu
from jax.experimental.pallas import tpu_sc as plsc

N = 10000
E = 160000
D = 256
H = 8
C = 32

NP = N + 16
ETOT = E + N
EPT = 10752
EPAD = 16 * EPT
PAD = EPAD - ETOT
CH = 128
NCH = EPT // CH
ACC_C = 144
NACC = N
RT = NACC // 16

_f32 = jnp.float32


@functools.lru_cache(maxsize=None)
def _make_edge_kernel():
    mesh = plsc.VectorSubcoreMesh(core_axis_name="c", subcore_axis_name="s")
    return functools.partial(
        pl.kernel,
        out_type=jax.ShapeDtypeStruct((2, NACC, ACC_C), _f32),
        mesh=mesh,
        compiler_params=pltpu.CompilerParams(
            needs_layout_passes=False, use_tc_tiling_on_sc=False),
        scratch_types=[
        pltpu.VMEM((CH,), jnp.int32),
        pltpu.VMEM((CH,), jnp.int32),
        pltpu.VMEM((CH,), jnp.int32),
        pltpu.VMEM((CH, 16), _f32),
        pltpu.VMEM((CH, 16), _f32),
        pltpu.VMEM((CH, CH), _f32),
        pltpu.VMEM((CH, ACC_C), _f32),
        pltpu.VMEM_SHARED((NACC, ACC_C), _f32),
        pltpu.SemaphoreType.DMA,
        pltpu.SemaphoreType.DMA,
        pltpu.SemaphoreType.DMA,
    ],
    )(_edge_body)


def _edge_body(T_hbm, XP_hbm, SRC3, DST3, ZR, out_hbm,
                 sidxc, didxsc, didxrc, g_s, g_d, xbuf, stage, acc,
                 sem0, sem1, sem2):
    c = lax.axis_index("c")
    t = lax.axis_index("s")
    off = c * NP

    pltpu.sync_copy(ZR, acc.at[pl.ds(t * RT, RT)])
    plsc.subcore_barrier()

    iota = lax.iota(jnp.int32, 16)

    def chunk_body(j, _):
        pltpu.sync_copy(SRC3.at[t, j], sidxc)
        pltpu.sync_copy(DST3.at[t, j], didxrc)
        for k in range(CH // 16):
            sl = pl.ds(k * 16, 16)
            sidxc[sl] = sidxc[sl] + off
            didxsc[sl] = didxrc[sl] + off
        cp_s = pltpu.async_copy(T_hbm.at[sidxc], g_s, sem0)
        cp_d = pltpu.async_copy(T_hbm.at[didxsc], g_d, sem1)
        cp_x = pltpu.async_copy(XP_hbm.at[sidxc], xbuf, sem2)
        cp_s.wait()
        cp_d.wait()
        for b in range(8):
            rowv = iota + (b * 16)
            for h in range(4):
                alsv = plsc.load_gather(g_s, [rowv, jnp.full((16,), h, jnp.int32)])
                aldv = plsc.load_gather(g_d, [rowv, jnp.full((16,), 4 + h, jnp.int32)])
                sv = plsc.load_gather(g_d, [rowv, jnp.full((16,), 8 + h, jnp.int32)])
                z = alsv + aldv
                e = jnp.maximum(z, 0.2 * z)
                ex = jnp.exp(e - sv)
                plsc.store_scatter(
                    stage, [rowv, jnp.full((16,), 128 + h, jnp.int32)], ex)
        cp_x.wait()

        def wb(kb, _):
            hh = kb // 4
            excol = jnp.full((16,), 128 + hh, jnp.int32)
            for b in range(8):
                rowv2 = iota + (b * 16)
                exv = plsc.load_gather(stage, [rowv2, excol])
                for k0 in range(8):
                    col = jnp.full((16,), kb * 8 + k0, jnp.int32)
                    xv = plsc.load_gather(xbuf, [rowv2, col])
                    plsc.store_scatter(stage, [rowv2, col], xv * exv)
            return 0

        lax.fori_loop(0, 16, wb, 0)
        pltpu.sync_copy(stage, acc.at[didxrc], add=True)
        return 0

    lax.fori_loop(0, NCH, chunk_body, 0)
    plsc.subcore_barrier()
    pltpu.sync_copy(acc.at[pl.ds(t * RT, RT)], out_hbm.at[c, pl.ds(t * RT, RT)])


BLK = 1000


def _accum_max(i, mx_ref, aa):
    m = jnp.broadcast_to(jnp.max(aa, axis=0, keepdims=True), (8, 16))

    @pl.when(i == 0)
    def _():
        mx_ref[...] = m

    @pl.when(i > 0)
    def _():
        mx_ref[...] = jnp.maximum(mx_ref[...], m)


def _k1_body(x_ref, Wi_ref, bi_ref, W_ref, Asd_ref,
             h_ref, xp_ref, aa_ref, mx_ref):
    i = pl.program_id(0)
    h = jax.nn.relu(
        jnp.dot(x_ref[...], Wi_ref[...], preferred_element_type=_f32)
        + bi_ref[...])
    h_ref[...] = h
    xp = jnp.dot(h, W_ref[...], preferred_element_type=_f32)
    xp_ref[...] = xp
    aa = jnp.dot(xp, Asd_ref[...], preferred_element_type=_f32)
    aa_ref[...] = aa
    _accum_max(i, mx_ref, aa)


def _kmid_body(out_ref, h_ref, bb_ref, g_ref, be_ref, W_ref, Asd_ref, Erep_ref,
               hn_ref, xp_ref, aa_ref, mx_ref):
    i = pl.program_id(0)
    raw0 = out_ref[0, :, 0:128]
    den0 = out_ref[0, :, 128:132]
    raw1 = out_ref[1, :, 0:128]
    den1 = out_ref[1, :, 128:132]
    den0e = jnp.dot(den0, Erep_ref[...], preferred_element_type=_f32)
    den1e = jnp.dot(den1, Erep_ref[...], preferred_element_type=_f32)
    gat = jnp.concatenate(
        [raw0 / (den0e + 1e-16), raw1 / (den1e + 1e-16)], axis=1) + bb_ref[...]
    mu = jnp.mean(gat, axis=1, keepdims=True)
    zc = gat - mu
    var = jnp.mean(zc * zc, axis=1, keepdims=True)
    hn = zc * lax.rsqrt(var + 1e-5) * g_ref[...] + be_ref[...]
    hn = jax.nn.relu(hn)
    hnew = h_ref[...] + hn
    hn_ref[...] = hnew
    xp = jnp.dot(hnew, W_ref[...], preferred_element_type=_f32)
    xp_ref[...] = xp
    aa = jnp.dot(xp, Asd_ref[...], preferred_element_type=_f32)
    aa_ref[...] = aa
    _accum_max(i, mx_ref, aa)


def _klast_body(out_ref, h_ref, bb_ref, g_ref, be_ref, Erep_ref, Wo_ref, bo_ref,
                y_ref):
    raw0 = out_ref[0, :, 0:128]
    den0 = out_ref[0, :, 128:132]
    raw1 = out_ref[1, :, 0:128]
    den1 = out_ref[1, :, 128:132]
    den0e = jnp.dot(den0, Erep_ref[...], preferred_element_type=_f32)
    den1e = jnp.dot(den1, Erep_ref[...], preferred_element_type=_f32)
    gat = jnp.concatenate(
        [raw0 / (den0e + 1e-16), raw1 / (den1e + 1e-16)], axis=1) + bb_ref[...]
    mu = jnp.mean(gat, axis=1, keepdims=True)
    zc = gat - mu
    var = jnp.mean(zc * zc, axis=1, keepdims=True)
    hn = zc * lax.rsqrt(var + 1e-5) * g_ref[...] + be_ref[...]
    hn = jax.nn.relu(hn)
    hnew = h_ref[...] + hn
    y_ref[...] = jnp.dot(hnew, Wo_ref[...], preferred_element_type=_f32) \
        + bo_ref[...]


def _full_spec():
    return pl.BlockSpec(None, lambda i: tuple())


def _k1(x, Wi, bi, W, Asd):
    return pl.pallas_call(
        _k1_body,
        grid=(N // BLK,),
        in_specs=[
            pl.BlockSpec((BLK, D), lambda i: (i, 0)),
            pl.BlockSpec((D, D), lambda i: (0, 0)),
            pl.BlockSpec((1, D), lambda i: (0, 0)),
            pl.BlockSpec((D, D), lambda i: (0, 0)),
            pl.BlockSpec((D, 16), lambda i: (0, 0)),
        ],
        out_specs=[
            pl.BlockSpec((BLK, D), lambda i: (i, 0)),
            pl.BlockSpec((BLK, D), lambda i: (i, 0)),
            pl.BlockSpec((BLK, 16), lambda i: (i, 0)),
            pl.BlockSpec((8, 16), lambda i: (0, 0)),
        ],
        out_shape=[
            jax.ShapeDtypeStruct((N, D), _f32),
            jax.ShapeDtypeStruct((N, D), _f32),
            jax.ShapeDtypeStruct((N, 16), _f32),
            jax.ShapeDtypeStruct((8, 16), _f32),
        ],
    )(x, Wi, bi.reshape(1, D), W, Asd)


def _kmid(out, h, bb, g, be, W, Asd, Erep):
    return pl.pallas_call(
        _kmid_body,
        grid=(N // BLK,),
        in_specs=[
            pl.BlockSpec((2, BLK, ACC_C), lambda i: (0, i, 0)),
            pl.BlockSpec((BLK, D), lambda i: (i, 0)),
            pl.BlockSpec((1, D), lambda i: (0, 0)),
            pl.BlockSpec((1, D), lambda i: (0, 0)),
            pl.BlockSpec((1, D), lambda i: (0, 0)),
            pl.BlockSpec((D, D), lambda i: (0, 0)),
            pl.BlockSpec((D, 16), lambda i: (0, 0)),
            pl.BlockSpec((4, 128), lambda i: (0, 0)),
        ],
        out_specs=[
            pl.BlockSpec((BLK, D), lambda i: (i, 0)),
            pl.BlockSpec((BLK, D), lambda i: (i, 0)),
            pl.BlockSpec((BLK, 16), lambda i: (i, 0)),
            pl.BlockSpec((8, 16), lambda i: (0, 0)),
        ],
        out_shape=[
            jax.ShapeDtypeStruct((N, D), _f32),
            jax.ShapeDtypeStruct((N, D), _f32),
            jax.ShapeDtypeStruct((N, 16), _f32),
            jax.ShapeDtypeStruct((8, 16), _f32),
        ],
    )(out, h, bb.reshape(1, D), g.reshape(1, D), be.reshape(1, D), W, Asd, Erep)


def _klast(out, h, bb, g, be, Erep, Wo, bo):
    return pl.pallas_call(
        _klast_body,
        grid=(N // BLK,),
        in_specs=[
            pl.BlockSpec((2, BLK, ACC_C), lambda i: (0, i, 0)),
            pl.BlockSpec((BLK, D), lambda i: (i, 0)),
            pl.BlockSpec((1, D), lambda i: (0, 0)),
            pl.BlockSpec((1, D), lambda i: (0, 0)),
            pl.BlockSpec((1, D), lambda i: (0, 0)),
            pl.BlockSpec((4, 128), lambda i: (0, 0)),
            pl.BlockSpec((D, 1), lambda i: (0, 0)),
            pl.BlockSpec((1, 1), lambda i: (0, 0)),
        ],
        out_specs=pl.BlockSpec((BLK, 1), lambda i: (i, 0)),
        out_shape=jax.ShapeDtypeStruct((N, 1), _f32),
    )(out, h, bb.reshape(1, D), g.reshape(1, D), be.reshape(1, D), Erep,
      Wo, bo.reshape(1, 1))


def _build_asd(a_s, a_d):
    eye = jnp.eye(H, dtype=_f32)
    As = (a_s[0][:, :, None] * eye[:, None, :]).reshape(H * C, H)
    Ad = (a_d[0][:, :, None] * eye[:, None, :]).reshape(H * C, H)
    return jnp.concatenate([As, Ad], axis=1)


def _build_tables(xp, aa, mx):
    am = mx[0, 0:8]
    dm = mx[0, 8:16]
    ssum = am + dm
    s = jnp.maximum(ssum, 0.2 * ssum)
    poison = jnp.concatenate(
        [jnp.full((16, 4), -1e30, _f32), jnp.zeros((16, 12), _f32)], axis=1)
    Ts = []
    for c in range(2):
        blk = jnp.concatenate(
            [aa[:, 4 * c:4 * c + 4],
             aa[:, 8 + 4 * c:8 + 4 * c + 4],
             jnp.broadcast_to(s[4 * c:4 * c + 4], (N, 4)),
             jnp.zeros((N, 4), _f32)], axis=1)
        Ts.append(jnp.concatenate([blk, poison], axis=0))
    T = jnp.concatenate(Ts, axis=0)
    xp2 = jnp.transpose(xp.reshape(N, 2, 128), (1, 0, 2))
    xp2 = jnp.pad(xp2, ((0, 0), (0, NP - N), (0, 0)))
    XP = xp2.reshape(2 * NP, 128)
    return T, XP


def kernel(x, edge_index, Wi, bi, W0, as0, ad0, bb0, g0, be0,
           W1, as1, ad1, bb1, g1, be1, W2, as2, ad2, bb2, g2, be2, Wo, bo):
    i32 = jnp.int32
    loops = jnp.arange(N, dtype=i32)
    srcp = jnp.concatenate(
        [edge_index[0].astype(i32), loops,
         N + (jnp.arange(PAD, dtype=i32) % 16)])
    dstp = jnp.concatenate(
        [edge_index[1].astype(i32), loops,
         jnp.arange(PAD, dtype=i32) % 16])
    SRC3 = srcp.reshape(16, NCH, CH)
    DST3 = dstp.reshape(16, NCH, CH)
    ZR = jnp.zeros((RT, ACC_C), _f32)
    Erep = jnp.kron(jnp.eye(4, dtype=_f32), jnp.ones((1, C), _f32))

    Asd = [_build_asd(as0, ad0), _build_asd(as1, ad1), _build_asd(as2, ad2)]
    Wl = [W0, W1, W2]
    bbl = [bb0, bb1, bb2]
    gl = [g0, g1, g2]
    bel = [be0, be1, be2]

    h, xp, aa, mx = _k1(x, Wi, bi, Wl[0], Asd[0])
    for l in range(3):
        T, XP = _build_tables(xp, aa, mx)
        out = _make_edge_kernel()(T, XP, SRC3, DST3, ZR)
        if l < 2:
            h, xp, aa, mx = _kmid(out, h, bbl[l], gl[l], bel[l],
                                  Wl[l + 1], Asd[l + 1], Erep)
        else:
            y = _klast(out, h, bbl[l], gl[l], bel[l], Erep, Wo, bo)
    return y


# speedup vs baseline: 29.3891x; 2.0698x over previous
import functools

import jax
import jax.numpy as jnp
from jax import lax
from jax.experimental import pallas as pl
from jax.experimental.pallas import tpu as pltpu
from jax.experimental.pallas import tpu_sc as plsc

N = 10000
E = 160000
D = 256
H = 8
C = 32

NP = N + 16
ETOT = E + N
CH = 64
BI = 8
NB = 21
NCH = NB * BI
EPT = NCH * CH
EPAD = 16 * EPT
PAD = EPAD - ETOT
ACC_C = 136
RT = N // 16

_f32 = jnp.float32


@functools.lru_cache(maxsize=None)
def _make_edge_kernel():
    mesh = plsc.VectorSubcoreMesh(core_axis_name="c", subcore_axis_name="s")
    return functools.partial(
        pl.kernel,
        out_type=jax.ShapeDtypeStruct((2, N, ACC_C), _f32),
        mesh=mesh,
        compiler_params=pltpu.CompilerParams(
            needs_layout_passes=False, use_tc_tiling_on_sc=False),
        scratch_types=[
            pltpu.VMEM((BI, CH), jnp.int32),
            pltpu.VMEM((BI, CH), jnp.int32),
            pltpu.VMEM((CH, 16), _f32),
            pltpu.VMEM((CH, 16), _f32),
            pltpu.VMEM((CH, 128), _f32),
            pltpu.VMEM((CH, ACC_C), _f32),
            pltpu.VMEM((CH, 16), _f32),
            pltpu.VMEM((CH, 16), _f32),
            pltpu.VMEM((CH, 128), _f32),
            pltpu.VMEM((CH, ACC_C), _f32),
            pltpu.VMEM((CH, 8), _f32),
            pltpu.VMEM_SHARED((N, ACC_C), _f32),
            pltpu.SemaphoreType.DMA,
            pltpu.SemaphoreType.DMA,
            pltpu.SemaphoreType.DMA,
            pltpu.SemaphoreType.DMA,
            pltpu.SemaphoreType.DMA,
            pltpu.SemaphoreType.DMA,
        ],
    )(_edge_body)


def _edge_body(T3, XP3, SRC4, DST4, ZR, out_hbm,
               sbuf, dbuf, g_s0, g_d0, xb0, st0, g_s1, g_d1, xb1, st1, exb,
               acc, se0, se1, se2, se3, se4, se5):
    c = lax.axis_index("c")
    t = lax.axis_index("s")

    pltpu.sync_copy(ZR, acc.at[pl.ds(t * RT, RT)])
    plsc.subcore_barrier()

    iota = lax.iota(jnp.int32, 16)
    gs = (g_s0, g_s1)
    gd = (g_d0, g_d1)
    xb = (xb0, xb1)
    st = (st0, st1)
    sems = ((se0, se1, se2), (se3, se4, se5))

    def issue(c0):
        s = c0 % 2
        idx = sbuf.at[c0]
        didx = dbuf.at[c0]
        return (pltpu.async_copy(T3.at[c].at[idx], gs[s], sems[s][0]),
                pltpu.async_copy(T3.at[c].at[didx], gd[s], sems[s][1]),
                pltpu.async_copy(XP3.at[c].at[idx], xb[s], sems[s][2]))

    def compute(c0, descs):
        s = c0 % 2
        d0, d1, d2 = descs
        d0.wait()
        d1.wait()
        g_s_, g_d_, xb_, st_ = gs[s], gd[s], xb[s], st[s]
        for b in range(CH // 16):
            rowv = iota + b * 16
            for h in range(4):
                alsv = plsc.load_gather(
                    g_s_, [rowv, jnp.full((16,), h, jnp.int32)])
                aldv = plsc.load_gather(
                    g_d_, [rowv, jnp.full((16,), 4 + h, jnp.int32)])
                sv = plsc.load_gather(
                    g_d_, [rowv, jnp.full((16,), 8 + h, jnp.int32)])
                z = alsv + aldv
                e = jnp.maximum(z, 0.2 * z)
                ex = jnp.exp(e - sv)
                plsc.store_scatter(
                    exb, [rowv, jnp.full((16,), h, jnp.int32)], ex)
                plsc.store_scatter(
                    st_, [rowv, jnp.full((16,), 128 + h, jnp.int32)], ex)
        d2.wait()

        def wrow(i, _):
            for u in range(2):
                r = i * 2 + u
                for h in range(4):
                    spl = plsc.load_gather(
                        exb, [jnp.full((16,), r, jnp.int32),
                              jnp.full((16,), h, jnp.int32)])
                    for k in range(2):
                        sl = pl.ds((h * 2 + k) * 16, 16)
                        st_[r, sl] = xb_[r, sl] * spl
            return 0

        lax.fori_loop(0, CH // 2, wrow, 0)
        pltpu.sync_copy(st_, acc.at[dbuf.at[c0]], add=True)

    def batch_body(ib, _):
        pltpu.sync_copy(SRC4.at[t, ib], sbuf)
        pltpu.sync_copy(DST4.at[t, ib], dbuf)
        descs = issue(0)
        for c0 in range(1, BI):
            nd = issue(c0)
            compute(c0 - 1, descs)
            descs = nd
        compute(BI - 1, descs)
        return 0

    lax.fori_loop(0, NB, batch_body, 0)
    plsc.subcore_barrier()
    pltpu.sync_copy(acc.at[pl.ds(t * RT, RT)], out_hbm.at[c, pl.ds(t * RT, RT)])


BLK = 1000


def _accum_max(i, mx_ref, aa):
    m = jnp.broadcast_to(jnp.max(aa, axis=0, keepdims=True), (8, 16))

    @pl.when(i == 0)
    def _():
        mx_ref[...] = m

    @pl.when(i > 0)
    def _():
        mx_ref[...] = jnp.maximum(mx_ref[...], m)


def _k1_body(x_ref, Wi_ref, bi_ref, W_ref, Asd_ref,
             h_ref, xp_ref, aa_ref, mx_ref):
    i = pl.program_id(0)
    h = jax.nn.relu(
        jnp.dot(x_ref[...], Wi_ref[...], preferred_element_type=_f32)
        + bi_ref[...])
    h_ref[...] = h
    xp = jnp.dot(h, W_ref[...], preferred_element_type=_f32)
    xp_ref[...] = xp
    aa = jnp.dot(xp, Asd_ref[...], preferred_element_type=_f32)
    aa_ref[...] = aa
    _accum_max(i, mx_ref, aa)


def _gat_post(out_ref, h_ref, bb_ref, g_ref, be_ref, Erep_ref):
    raw0 = out_ref[0, :, 0:128]
    den0 = out_ref[0, :, 128:132]
    raw1 = out_ref[1, :, 0:128]
    den1 = out_ref[1, :, 128:132]
    den0e = jnp.dot(den0, Erep_ref[...], preferred_element_type=_f32)
    den1e = jnp.dot(den1, Erep_ref[...], preferred_element_type=_f32)
    gat = jnp.concatenate(
        [raw0 / (den0e + 1e-16), raw1 / (den1e + 1e-16)], axis=1) + bb_ref[...]
    mu = jnp.mean(gat, axis=1, keepdims=True)
    zc = gat - mu
    var = jnp.mean(zc * zc, axis=1, keepdims=True)
    hn = zc * lax.rsqrt(var + 1e-5) * g_ref[...] + be_ref[...]
    hn = jax.nn.relu(hn)
    return h_ref[...] + hn


def _kmid_body(out_ref, h_ref, bb_ref, g_ref, be_ref, W_ref, Asd_ref, Erep_ref,
               hn_ref, xp_ref, aa_ref, mx_ref):
    i = pl.program_id(0)
    hnew = _gat_post(out_ref, h_ref, bb_ref, g_ref, be_ref, Erep_ref)
    hn_ref[...] = hnew
    xp = jnp.dot(hnew, W_ref[...], preferred_element_type=_f32)
    xp_ref[...] = xp
    aa = jnp.dot(xp, Asd_ref[...], preferred_element_type=_f32)
    aa_ref[...] = aa
    _accum_max(i, mx_ref, aa)


def _klast_body(out_ref, h_ref, bb_ref, g_ref, be_ref, Erep_ref, Wo_ref,
                bo_ref, y_ref):
    hnew = _gat_post(out_ref, h_ref, bb_ref, g_ref, be_ref, Erep_ref)
    y_ref[...] = jnp.dot(hnew, Wo_ref[...], preferred_element_type=_f32) \
        + bo_ref[...]


def _k1(x, Wi, bi, W, Asd):
    return pl.pallas_call(
        _k1_body,
        grid=(N // BLK,),
        in_specs=[
            pl.BlockSpec((BLK, D), lambda i: (i, 0)),
            pl.BlockSpec((D, D), lambda i: (0, 0)),
            pl.BlockSpec((1, D), lambda i: (0, 0)),
            pl.BlockSpec((D, D), lambda i: (0, 0)),
            pl.BlockSpec((D, 16), lambda i: (0, 0)),
        ],
        out_specs=[
            pl.BlockSpec((BLK, D), lambda i: (i, 0)),
            pl.BlockSpec((BLK, D), lambda i: (i, 0)),
            pl.BlockSpec((BLK, 16), lambda i: (i, 0)),
            pl.BlockSpec((8, 16), lambda i: (0, 0)),
        ],
        out_shape=[
            jax.ShapeDtypeStruct((N, D), _f32),
            jax.ShapeDtypeStruct((N, D), _f32),
            jax.ShapeDtypeStruct((N, 16), _f32),
            jax.ShapeDtypeStruct((8, 16), _f32),
        ],
    )(x, Wi, bi.reshape(1, D), W, Asd)


def _kmid(out, h, bb, g, be, W, Asd, Erep):
    return pl.pallas_call(
        _kmid_body,
        grid=(N // BLK,),
        in_specs=[
            pl.BlockSpec((2, BLK, ACC_C), lambda i: (0, i, 0)),
            pl.BlockSpec((BLK, D), lambda i: (i, 0)),
            pl.BlockSpec((1, D), lambda i: (0, 0)),
            pl.BlockSpec((1, D), lambda i: (0, 0)),
            pl.BlockSpec((1, D), lambda i: (0, 0)),
            pl.BlockSpec((D, D), lambda i: (0, 0)),
            pl.BlockSpec((D, 16), lambda i: (0, 0)),
            pl.BlockSpec((4, 128), lambda i: (0, 0)),
        ],
        out_specs=[
            pl.BlockSpec((BLK, D), lambda i: (i, 0)),
            pl.BlockSpec((BLK, D), lambda i: (i, 0)),
            pl.BlockSpec((BLK, 16), lambda i: (i, 0)),
            pl.BlockSpec((8, 16), lambda i: (0, 0)),
        ],
        out_shape=[
            jax.ShapeDtypeStruct((N, D), _f32),
            jax.ShapeDtypeStruct((N, D), _f32),
            jax.ShapeDtypeStruct((N, 16), _f32),
            jax.ShapeDtypeStruct((8, 16), _f32),
        ],
    )(out, h, bb.reshape(1, D), g.reshape(1, D), be.reshape(1, D), W, Asd,
      Erep)


def _klast(out, h, bb, g, be, Erep, Wo, bo):
    return pl.pallas_call(
        _klast_body,
        grid=(N // BLK,),
        in_specs=[
            pl.BlockSpec((2, BLK, ACC_C), lambda i: (0, i, 0)),
            pl.BlockSpec((BLK, D), lambda i: (i, 0)),
            pl.BlockSpec((1, D), lambda i: (0, 0)),
            pl.BlockSpec((1, D), lambda i: (0, 0)),
            pl.BlockSpec((1, D), lambda i: (0, 0)),
            pl.BlockSpec((4, 128), lambda i: (0, 0)),
            pl.BlockSpec((D, 1), lambda i: (0, 0)),
            pl.BlockSpec((1, 1), lambda i: (0, 0)),
        ],
        out_specs=pl.BlockSpec((BLK, 1), lambda i: (i, 0)),
        out_shape=jax.ShapeDtypeStruct((N, 1), _f32),
    )(out, h, bb.reshape(1, D), g.reshape(1, D), be.reshape(1, D), Erep,
      Wo, bo.reshape(1, 1))


def _build_asd(a_s, a_d):
    eye = jnp.eye(H, dtype=_f32)
    As = (a_s[0][:, :, None] * eye[:, None, :]).reshape(H * C, H)
    Ad = (a_d[0][:, :, None] * eye[:, None, :]).reshape(H * C, H)
    return jnp.concatenate([As, Ad], axis=1)


def _build_tables(xp, aa, mx):
    am = mx[0, 0:8]
    dm = mx[0, 8:16]
    ssum = am + dm
    s = jnp.maximum(ssum, 0.2 * ssum)
    poison = jnp.concatenate(
        [jnp.full((16, 4), -1e30, _f32), jnp.zeros((16, 12), _f32)], axis=1)
    Ts = []
    for cc in range(2):
        blk = jnp.concatenate(
            [aa[:, 4 * cc:4 * cc + 4],
             aa[:, 8 + 4 * cc:8 + 4 * cc + 4],
             jnp.broadcast_to(s[4 * cc:4 * cc + 4], (N, 4)),
             jnp.zeros((N, 4), _f32)], axis=1)
        Ts.append(jnp.concatenate([blk, poison], axis=0))
    T = jnp.stack(Ts, axis=0)
    xp2 = jnp.transpose(xp.reshape(N, 2, 128), (1, 0, 2))
    XP = jnp.pad(xp2, ((0, 0), (0, NP - N), (0, 0)))
    return T, XP


def kernel(x, edge_index, Wi, bi, W0, as0, ad0, bb0, g0, be0,
           W1, as1, ad1, bb1, g1, be1, W2, as2, ad2, bb2, g2, be2, Wo, bo):
    i32 = jnp.int32
    loops = jnp.arange(N, dtype=i32)
    srcp = jnp.concatenate(
        [edge_index[0].astype(i32), loops,
         N + (jnp.arange(PAD, dtype=i32) % 16)])
    dstp = jnp.concatenate(
        [edge_index[1].astype(i32), loops,
         jnp.arange(PAD, dtype=i32) % 16])
    SRC4 = srcp.reshape(16, NB, BI, CH)
    DST4 = dstp.reshape(16, NB, BI, CH)
    ZR = jnp.zeros((RT, ACC_C), _f32)
    Erep = jnp.kron(jnp.eye(4, dtype=_f32), jnp.ones((1, C), _f32))

    Asd = [_build_asd(as0, ad0), _build_asd(as1, ad1), _build_asd(as2, ad2)]
    Wl = [W0, W1, W2]
    bbl = [bb0, bb1, bb2]
    gl = [g0, g1, g2]
    bel = [be0, be1, be2]

    h, xp, aa, mx = _k1(x, Wi, bi, Wl[0], Asd[0])
    for l in range(3):
        T, XP = _build_tables(xp, aa, mx)
        out = _make_edge_kernel()(T, XP, SRC4, DST4, ZR)
        if l < 2:
            h, xp, aa, mx = _kmid(out, h, bbl[l], gl[l], bel[l],
                                  Wl[l + 1], Asd[l + 1], Erep)
        else:
            y = _klast(out, h, bbl[l], gl[l], bel[l], Erep, Wo, bo)
    return y
